# Initial kernel scaffold; baseline (speedup 1.0000x reference)
import jax
import jax.numpy as jnp
from jax.experimental import pallas as pl


def kernel(node_state, edge_index, point_enc, edge_weight, point_enc_w, norm_w, norm_b, edge_enc, lin_W, lin_b, conv_ln_w, conv_ln_b):
    raise NotImplementedError("write your pallas kernel here")


# speedup vs baseline: 3.1617x; 3.1617x over previous
import functools
import math

import jax
import jax.numpy as jnp
from jax import lax
from jax.experimental import pallas as pl
from jax.experimental.pallas import tpu as pltpu
from jax.experimental.pallas import tpu_sc as plsc

N = 10000
D = 128
E = 320000
NENT = 100
NB = 10
BLK = N // NB


def _ln(x, w, b, eps=1e-5):
    mu = jnp.mean(x, axis=-1, keepdims=True)
    var = jnp.mean((x - mu) ** 2, axis=-1, keepdims=True)
    return (x - mu) * lax.rsqrt(var + eps) * w + b


def _dense_body(x_ref, pe_ref, hist_ref, W_ref, linW_ref, nw_ref, nb_ref,
                cw_ref, cb_ref, lb_ref, ee_ref,
                h_ref, g_ref, hd_ref, dinv_ref):
    x = x_ref[...]
    pe = pe_ref[0]
    xb = x.astype(jnp.bfloat16)

    def body(e, acc):
        m = (pe == e)
        xm = jnp.where(m, xb, jnp.zeros_like(xb))
        w = W_ref[e].astype(jnp.bfloat16)
        return acc + lax.dot_general(
            xm, w, (((1,), (0,)), ((), ())),
            preferred_element_type=jnp.float32)

    acc = lax.fori_loop(0, NENT, body, jnp.zeros((BLK, D), jnp.float32))
    ns = _ln(acc, nw_ref[0], nb_ref[0])
    ns = jnp.maximum(ns, 0.0)
    h = lax.dot_general(
        ns.astype(jnp.bfloat16), linW_ref[...].astype(jnp.bfloat16),
        (((1,), (1,)), ((), ())), preferred_element_type=jnp.float32)
    h = h + lb_ref[0]
    h = _ln(h, cw_ref[0], cb_ref[0])
    deg = 1.0 + jnp.sum(hist_ref[0], axis=-1, keepdims=True)
    dinv = lax.rsqrt(deg)
    h_ref[...] = h
    g_ref[...] = h * dinv * ee_ref[...]
    hd_ref[...] = h * (dinv * dinv)
    dinv_ref[0] = dinv


def _tc_dense(x, pe3, hist, W, linW, nw, nb, cw, cb, lb, ee):
    grid = (NB,)
    return pl.pallas_call(
        _dense_body,
        grid=grid,
        in_specs=[
            pl.BlockSpec((BLK, D), lambda i: (i, 0)),
            pl.BlockSpec((1, BLK, 1), lambda i: (i, 0, 0)),
            pl.BlockSpec((1, BLK, 32), lambda i: (i, 0, 0)),
            pl.BlockSpec((NENT, D, D), lambda i: (0, 0, 0)),
            pl.BlockSpec((D, D), lambda i: (0, 0)),
            pl.BlockSpec((1, D), lambda i: (0, 0)),
            pl.BlockSpec((1, D), lambda i: (0, 0)),
            pl.BlockSpec((1, D), lambda i: (0, 0)),
            pl.BlockSpec((1, D), lambda i: (0, 0)),
            pl.BlockSpec((1, D), lambda i: (0, 0)),
            pl.BlockSpec((1, D), lambda i: (0, 0)),
        ],
        out_specs=[
            pl.BlockSpec((BLK, D), lambda i: (i, 0)),
            pl.BlockSpec((BLK, D), lambda i: (i, 0)),
            pl.BlockSpec((BLK, D), lambda i: (i, 0)),
            pl.BlockSpec((1, BLK, 1), lambda i: (i, 0, 0)),
        ],
        out_shape=[
            jax.ShapeDtypeStruct((N, D), jnp.float32),
            jax.ShapeDtypeStruct((N, D), jnp.float32),
            jax.ShapeDtypeStruct((N, D), jnp.float32),
            jax.ShapeDtypeStruct((NB, BLK, 1), jnp.float32),
        ],
    )(x, pe3, hist, W, linW, nw, nb, cw, cb, lb, ee)


def _combine_body(acc_ref, hd_ref, dinv_ref, out_ref):
    s = acc_ref[0] + acc_ref[1]
    out_ref[...] = jnp.maximum(s * dinv_ref[0] + hd_ref[...], 0.0)


def _tc_combine(acc, hd, dinv3):
    return pl.pallas_call(
        _combine_body,
        grid=(NB,),
        in_specs=[
            pl.BlockSpec((2, BLK, D), lambda i: (0, i, 0)),
            pl.BlockSpec((BLK, D), lambda i: (i, 0)),
            pl.BlockSpec((1, BLK, 1), lambda i: (i, 0, 0)),
        ],
        out_specs=pl.BlockSpec((BLK, D), lambda i: (i, 0)),
        out_shape=jax.ShapeDtypeStruct((N, D), jnp.float32),
    )(acc, hd, dinv3)


def kernel(node_state, edge_index, point_enc, edge_weight, point_enc_w,
           norm_w, norm_b, edge_enc, lin_W, lin_b, conv_ln_w, conv_ln_b):
    x = node_state.reshape(N, D)
    pe3 = point_enc.reshape(NB, BLK, 1).astype(jnp.int32)
    row = edge_index[0, 0].astype(jnp.int32)
    col = edge_index[0, 1].astype(jnp.int32)
    ew = edge_weight.reshape(E)
    ee = jnp.broadcast_to(edge_enc.reshape(1, 1), (1, D))

    hist1 = jnp.zeros((N,), jnp.float32).at[col].add(1.0)
    hist32 = jnp.zeros((32, N), jnp.float32).at[0].set(hist1)
    hist = hist32.T.reshape(NB, BLK, 32)

    h, g, hd, dinv3 = _tc_dense(
        x, pe3, hist, point_enc_w, lin_W,
        norm_w.reshape(1, D), norm_b.reshape(1, D),
        conv_ln_w.reshape(1, D), conv_ln_b.reshape(1, D),
        lin_b.reshape(1, D), ee)

    msg = ew[:, None] * g[row]
    acc1 = jnp.zeros((N, D), jnp.float32).at[col].add(msg)
    acc = jnp.stack([acc1, jnp.zeros_like(acc1)], axis=0)

    out = _tc_combine(acc, hd, dinv3)
    return out.reshape(1, N, D)


# speedup vs baseline: 9.9889x; 3.1593x over previous
import functools
import math

import jax
import jax.numpy as jnp
from jax import lax
from jax.experimental import pallas as pl
from jax.experimental.pallas import tpu as pltpu
from jax.experimental.pallas import tpu_sc as plsc

N = 10000
D = 128
E = 320000
NENT = 100
NB = 10
BLK = N // NB


def _ln(x, w, b, eps=1e-5):
    mu = jnp.mean(x, axis=-1, keepdims=True)
    var = jnp.mean((x - mu) ** 2, axis=-1, keepdims=True)
    return (x - mu) * lax.rsqrt(var + eps) * w + b


NC, NS = 2, 16
NW = NC * NS
EPW = E // NW
HCH = 2000

def _hist_body(col_hbm, out_hbm, colv, hist_v, sem):
    wid = lax.axis_index("s") * NC + lax.axis_index("c")
    zero16 = jnp.zeros((16,), jnp.float32)
    one16 = jnp.full((16,), 1.0, jnp.float32)

    def zbody(i, _):
        hist_v[pl.ds(pl.multiple_of(i * 16, 16), 16)] = zero16
        return 0
    lax.fori_loop(0, N // 16, zbody, 0)

    base = wid * EPW

    def chunk(c, _):
        pltpu.sync_copy(col_hbm.at[pl.ds(base + c * HCH, HCH)], colv)

        def ibody(i, _):
            idx16 = colv[pl.ds(pl.multiple_of(i * 16, 16), 16)]
            plsc.addupdate_scatter(hist_v, [idx16], one16)
            return 0
        lax.fori_loop(0, HCH // 16, ibody, 0)
        return 0
    lax.fori_loop(0, EPW // HCH, chunk, 0)

    pltpu.sync_copy(hist_v, out_hbm.at[wid])


def _sc_hist(col):
    mesh = plsc.VectorSubcoreMesh(core_axis_name="c", subcore_axis_name="s")
    f = pl.kernel(
        _hist_body,
        out_type=jax.ShapeDtypeStruct((NW, N), jnp.float32),
        mesh=mesh,
        compiler_params=pltpu.CompilerParams(needs_layout_passes=False),
        scratch_types=[
            pltpu.VMEM((HCH,), jnp.int32),
            pltpu.VMEM((N,), jnp.float32),
            pltpu.SemaphoreType.DMA,
        ],
    )
    return f(col)


CHUNK = 128
NCH = E // CHUNK
RPT = N // NS


def _lane_bcast(v, l):
    dnums = lax.GatherDimensionNumbers(
        offset_dims=(), collapsed_slice_dims=(0,), start_index_map=(0,))
    idx = jnp.full((16, 1), l, jnp.int32)
    return lax.gather(v, idx, dnums, (1,),
                      mode=lax.GatherScatterMode.PROMISE_IN_BOUNDS)


def _scat_body(g_hbm, row_hbm, col_hbm, ew_hbm, zeros_hbm, out_hbm,
               idx_row, idx_col, ew_v, rows_v, acc_sh, sem):
    cid = lax.axis_index("c")
    sid = lax.axis_index("s")
    wid = sid * NC + cid

    pltpu.sync_copy(zeros_hbm, acc_sh.at[pl.ds(sid * RPT, RPT)])
    plsc.subcore_barrier()

    def chunk(k, _):
        c = wid + NW * k

        @pl.when(c < NCH)
        def _():
            base = c * CHUNK
            pltpu.sync_copy(row_hbm.at[pl.ds(base, CHUNK)], idx_row)
            pltpu.sync_copy(col_hbm.at[pl.ds(base, CHUNK)], idx_col)
            pltpu.sync_copy(ew_hbm.at[pl.ds(base, CHUNK)], ew_v)
            pltpu.async_copy(g_hbm.at[idx_row], rows_v, sem).wait()
            for j in range(CHUNK // 16):
                wv = ew_v[pl.ds(j * 16, 16)]
                for l in range(16):
                    cb = _lane_bcast(wv, l)
                    e = j * 16 + l
                    for q in range(D // 16):
                        sl = pl.ds(q * 16, 16)
                        rows_v[e, sl] = rows_v[e, sl] * cb
            pltpu.sync_copy(rows_v, acc_sh.at[idx_col], add=True)
        return 0

    lax.fori_loop(0, (NCH + NW - 1) // NW, chunk, 0)

    plsc.subcore_barrier()
    pltpu.sync_copy(acc_sh.at[pl.ds(sid * RPT, RPT)], out_hbm.at[cid, sid])


def _sc_scatter(g, row, col, ew, zeros):
    mesh = plsc.VectorSubcoreMesh(core_axis_name="c", subcore_axis_name="s")
    f = pl.kernel(
        _scat_body,
        out_type=jax.ShapeDtypeStruct((NC, NS, RPT, D), jnp.float32),
        mesh=mesh,
        compiler_params=pltpu.CompilerParams(needs_layout_passes=False),
        scratch_types=[
            pltpu.VMEM((CHUNK,), jnp.int32),
            pltpu.VMEM((CHUNK,), jnp.int32),
            pltpu.VMEM((CHUNK,), jnp.float32),
            pltpu.VMEM((CHUNK, D), jnp.float32),
            pltpu.VMEM_SHARED((N, D), jnp.float32),
            pltpu.SemaphoreType.DMA,
        ],
    )
    return f(g, row, col, ew, zeros)


def _dense_body(x_ref, pe_ref, hist_ref, W_ref, linW_ref, nw_ref, nb_ref,
                cw_ref, cb_ref, lb_ref, ee_ref,
                h_ref, g_ref, hd_ref, dinv_ref):
    x = x_ref[...]
    pe = pe_ref[0]
    xb = x.astype(jnp.bfloat16)

    def body(e, acc):
        m = (pe == e)
        xm = jnp.where(m, xb, jnp.zeros_like(xb))
        w = W_ref[e].astype(jnp.bfloat16)
        return acc + lax.dot_general(
            xm, w, (((1,), (0,)), ((), ())),
            preferred_element_type=jnp.float32)

    acc = lax.fori_loop(0, NENT, body, jnp.zeros((BLK, D), jnp.float32))
    ns = _ln(acc, nw_ref[0], nb_ref[0])
    ns = jnp.maximum(ns, 0.0)
    h = lax.dot_general(
        ns.astype(jnp.bfloat16), linW_ref[...].astype(jnp.bfloat16),
        (((1,), (1,)), ((), ())), preferred_element_type=jnp.float32)
    h = h + lb_ref[0]
    h = _ln(h, cw_ref[0], cb_ref[0])
    deg = 1.0 + jnp.sum(hist_ref[0], axis=-1, keepdims=True)
    dinv = lax.rsqrt(deg)
    h_ref[...] = h
    g_ref[...] = h * dinv * ee_ref[...]
    hd_ref[...] = h * (dinv * dinv)
    dinv_ref[0] = dinv


def _tc_dense(x, pe3, hist, W, linW, nw, nb, cw, cb, lb, ee):
    grid = (NB,)
    return pl.pallas_call(
        _dense_body,
        grid=grid,
        in_specs=[
            pl.BlockSpec((BLK, D), lambda i: (i, 0)),
            pl.BlockSpec((1, BLK, 1), lambda i: (i, 0, 0)),
            pl.BlockSpec((1, BLK, 32), lambda i: (i, 0, 0)),
            pl.BlockSpec((NENT, D, D), lambda i: (0, 0, 0)),
            pl.BlockSpec((D, D), lambda i: (0, 0)),
            pl.BlockSpec((1, D), lambda i: (0, 0)),
            pl.BlockSpec((1, D), lambda i: (0, 0)),
            pl.BlockSpec((1, D), lambda i: (0, 0)),
            pl.BlockSpec((1, D), lambda i: (0, 0)),
            pl.BlockSpec((1, D), lambda i: (0, 0)),
            pl.BlockSpec((1, D), lambda i: (0, 0)),
        ],
        out_specs=[
            pl.BlockSpec((BLK, D), lambda i: (i, 0)),
            pl.BlockSpec((BLK, D), lambda i: (i, 0)),
            pl.BlockSpec((BLK, D), lambda i: (i, 0)),
            pl.BlockSpec((1, BLK, 1), lambda i: (i, 0, 0)),
        ],
        out_shape=[
            jax.ShapeDtypeStruct((N, D), jnp.float32),
            jax.ShapeDtypeStruct((N, D), jnp.float32),
            jax.ShapeDtypeStruct((N, D), jnp.float32),
            jax.ShapeDtypeStruct((NB, BLK, 1), jnp.float32),
        ],
    )(x, pe3, hist, W, linW, nw, nb, cw, cb, lb, ee)


def _combine_body(acc_ref, hd_ref, dinv_ref, out_ref):
    s = acc_ref[0] + acc_ref[1]
    out_ref[...] = jnp.maximum(s * dinv_ref[0] + hd_ref[...], 0.0)


def _tc_combine(acc, hd, dinv3):
    return pl.pallas_call(
        _combine_body,
        grid=(NB,),
        in_specs=[
            pl.BlockSpec((2, BLK, D), lambda i: (0, i, 0)),
            pl.BlockSpec((BLK, D), lambda i: (i, 0)),
            pl.BlockSpec((1, BLK, 1), lambda i: (i, 0, 0)),
        ],
        out_specs=pl.BlockSpec((BLK, D), lambda i: (i, 0)),
        out_shape=jax.ShapeDtypeStruct((N, D), jnp.float32),
    )(acc, hd, dinv3)


def kernel(node_state, edge_index, point_enc, edge_weight, point_enc_w,
           norm_w, norm_b, edge_enc, lin_W, lin_b, conv_ln_w, conv_ln_b):
    x = node_state.reshape(N, D)
    pe3 = point_enc.reshape(NB, BLK, 1).astype(jnp.int32)
    row = edge_index[0, 0].astype(jnp.int32)
    col = edge_index[0, 1].astype(jnp.int32)
    ew = edge_weight.reshape(E)
    ee = jnp.broadcast_to(edge_enc.reshape(1, 1), (1, D))

    hist32 = _sc_hist(col)
    hist = hist32.T.reshape(NB, BLK, 32)

    h, g, hd, dinv3 = _tc_dense(
        x, pe3, hist, point_enc_w, lin_W,
        norm_w.reshape(1, D), norm_b.reshape(1, D),
        conv_ln_w.reshape(1, D), conv_ln_b.reshape(1, D),
        lin_b.reshape(1, D), ee)

    zeros = jnp.zeros((RPT, D), jnp.float32)
    acc = _sc_scatter(g, row, col, ew, zeros).reshape(NC, N, D)

    out = _tc_combine(acc, hd, dinv3)
    return out.reshape(1, N, D)
